# baseline (speedup 1.0000x reference)
import jax
import jax.numpy as jnp
import numpy as np
from jax.experimental import pallas as pl
from jax.experimental.pallas import tpu as pltpu

K = 2
ENC = 1024
IN_DIM = 2048
OUT_DIM = 4096
HID = 512
BLK_M = 256
SEQ_BLK = K * BLK_M

_SEL = np.zeros((2 * BLK_M, SEQ_BLK), dtype=np.float32)
for _r in range(BLK_M):
    _SEL[_r, K * _r] = 1.0
    _SEL[BLK_M + _r, K * _r + 1] = 1.0


def _fused_kernel(x_ref, s_ref, g1_ref, b1_ref, w12_ref, w3_ref, g2_ref,
                  b2_ref, out_ref, w12g_ref, w3b_ref, hb_ref, cs_ref):
    @pl.when(jnp.logical_and(pl.program_id(0) == 0, pl.program_id(1) == 0))
    def _precompute():
        w12g_ref[...] = (w12_ref[...] * g1_ref[...]).astype(jnp.bfloat16)
        w3b_ref[...] = w3_ref[...].astype(jnp.bfloat16)
        hb_ref[...] = jax.lax.dot_general(
            b1_ref[...], w12_ref[...], (((1,), (1,)), ((), ())),
            preferred_element_type=jnp.float32)
        cs_ref[...] = jax.lax.dot_general(
            g1_ref[...], w12_ref[...], (((1,), (1,)), ((), ())),
            preferred_element_type=jnp.float32)

    xb = x_ref[0].astype(jnp.bfloat16)
    ext = jax.lax.dot_general(s_ref[...], xb, (((1,), (0,)), ((), ())),
                              preferred_element_type=jnp.float32)
    extb = ext.astype(jnp.bfloat16)
    sqb = (ext * ext).astype(jnp.bfloat16)
    ones1 = jnp.ones((ENC, 128), dtype=jnp.bfloat16)
    s1 = jax.lax.dot_general(extb, ones1, (((1,), (0,)), ((), ())),
                             preferred_element_type=jnp.float32)
    s2 = jax.lax.dot_general(sqb, ones1, (((1,), (0,)), ((), ())),
                             preferred_element_type=jnp.float32)
    mean = (s1[:BLK_M, :1] + s1[BLK_M:, :1]) * (1.0 / IN_DIM)
    ex2 = (s2[:BLK_M, :1] + s2[BLK_M:, :1]) * (1.0 / IN_DIM)
    var = ex2 - mean * mean
    rs = jax.lax.rsqrt(var + 1e-6)
    h_raw = (jax.lax.dot_general(extb[:BLK_M], w12g_ref[:, :ENC],
                                 (((1,), (1,)), ((), ())),
                                 preferred_element_type=jnp.float32)
             + jax.lax.dot_general(extb[BLK_M:], w12g_ref[:, ENC:],
                                   (((1,), (1,)), ((), ())),
                                   preferred_element_type=jnp.float32))
    h = (h_raw - mean * cs_ref[...]) * rs + hb_ref[...]
    gate = h[:, :HID]
    val = h[:, HID:]
    act = (gate * jax.nn.sigmoid(gate) * val).astype(jnp.bfloat16)
    y = jax.lax.dot_general(act, w3b_ref[...], (((1,), (1,)), ((), ())),
                            preferred_element_type=jnp.float32)
    yb = y.astype(jnp.bfloat16)
    ysqb = (y * y).astype(jnp.bfloat16)
    ones2 = jnp.ones((OUT_DIM, 128), dtype=jnp.bfloat16)
    t1 = jax.lax.dot_general(yb, ones2, (((1,), (0,)), ((), ())),
                             preferred_element_type=jnp.float32)
    t2 = jax.lax.dot_general(ysqb, ones2, (((1,), (0,)), ((), ())),
                             preferred_element_type=jnp.float32)
    mean2 = t1[:, :1] * (1.0 / OUT_DIM)
    var2 = t2[:, :1] * (1.0 / OUT_DIM) - mean2 * mean2
    rs2 = jax.lax.rsqrt(var2 + 1e-6)
    out_ref[0] = ((y - mean2) * rs2) * g2_ref[...] + b2_ref[...]


def kernel(x, ln_pre_g, ln_pre_b, w12, w3, router_w, router_b, ln_post_g,
           ln_post_b):
    b, s, d = x.shape
    m = s // K
    nb = m // BLK_M
    sel = jnp.asarray(_SEL, dtype=jnp.bfloat16)
    out = pl.pallas_call(
        _fused_kernel,
        grid=(b, nb),
        in_specs=[
            pl.BlockSpec((1, SEQ_BLK, ENC), lambda i, j: (i, j, 0)),
            pl.BlockSpec((2 * BLK_M, SEQ_BLK), lambda i, j: (0, 0)),
            pl.BlockSpec((1, IN_DIM), lambda i, j: (0, 0)),
            pl.BlockSpec((1, IN_DIM), lambda i, j: (0, 0)),
            pl.BlockSpec((2 * HID, IN_DIM), lambda i, j: (0, 0)),
            pl.BlockSpec((OUT_DIM, HID), lambda i, j: (0, 0)),
            pl.BlockSpec((1, OUT_DIM), lambda i, j: (0, 0)),
            pl.BlockSpec((1, OUT_DIM), lambda i, j: (0, 0)),
        ],
        out_specs=pl.BlockSpec((1, BLK_M, OUT_DIM), lambda i, j: (i, j, 0)),
        out_shape=jax.ShapeDtypeStruct((b, m, OUT_DIM), jnp.float32),
        scratch_shapes=[
            pltpu.VMEM((2 * HID, IN_DIM), jnp.bfloat16),
            pltpu.VMEM((OUT_DIM, HID), jnp.bfloat16),
            pltpu.VMEM((1, 2 * HID), jnp.float32),
            pltpu.VMEM((1, 2 * HID), jnp.float32),
        ],
    )(x, sel, ln_pre_g.reshape(1, -1), ln_pre_b.reshape(1, -1), w12, w3,
      ln_post_g.reshape(1, -1), ln_post_b.reshape(1, -1))
    aux_loss = jnp.zeros((), jnp.float32)
    return (out, aux_loss)
